# baseline (speedup 1.0000x reference)
import jax
import jax.numpy as jnp
from jax.experimental import pallas as pl

_TAR_BLK = 256
_SRC_BLK = 1000


def _tar_mm_kernel(hg_ref, x_ref, o_ref):
    o_ref[...] = jnp.dot(hg_ref[...], x_ref[...],
                         preferred_element_type=jnp.float32)


def _tar_mm(hg_tar, x):
    h, n = hg_tar.shape
    d = x.shape[1]
    return pl.pallas_call(
        _tar_mm_kernel,
        grid=(h // _TAR_BLK,),
        in_specs=[
            pl.BlockSpec((_TAR_BLK, n), lambda i: (i, 0)),
            pl.BlockSpec((n, d), lambda i: (0, 0)),
        ],
        out_specs=pl.BlockSpec((_TAR_BLK, d), lambda i: (i, 0)),
        out_shape=jax.ShapeDtypeStruct((h, d), jnp.float32),
    )(hg_tar, x)


def _src_mm_kernel(hg_ref, t_ref, xprev_ref, o_ref):
    s = jnp.dot(hg_ref[...], t_ref[...], preferred_element_type=jnp.float32)
    o_ref[...] = jnp.maximum(s, 0.0) + xprev_ref[...]


def _src_mm(hg_src, t, xprev):
    n, h = hg_src.shape
    d = t.shape[1]
    return pl.pallas_call(
        _src_mm_kernel,
        grid=(n // _SRC_BLK,),
        in_specs=[
            pl.BlockSpec((_SRC_BLK, h), lambda i: (i, 0)),
            pl.BlockSpec((h, d), lambda i: (0, 0)),
            pl.BlockSpec((_SRC_BLK, d), lambda i: (i, 0)),
        ],
        out_specs=pl.BlockSpec((_SRC_BLK, d), lambda i: (i, 0)),
        out_shape=jax.ShapeDtypeStruct((n, d), jnp.float32),
    )(hg_src, t, xprev)


def _src_final_kernel(hg_ref, t_ref, x1_ref, x0_ref, att_ref, o_ref):
    a = att_ref[0, :]
    e = jnp.exp(a - jnp.max(a))
    w = e / jnp.sum(e)
    s = jnp.dot(hg_ref[...], t_ref[...], preferred_element_type=jnp.float32)
    o_ref[...] = (w[0] * x0_ref[...] + (w[1] + w[2]) * x1_ref[...]
                  + w[2] * jnp.maximum(s, 0.0))


def _src_final(hg_src, t, x1, x0, att):
    n, h = hg_src.shape
    d = t.shape[1]
    return pl.pallas_call(
        _src_final_kernel,
        grid=(n // _SRC_BLK,),
        in_specs=[
            pl.BlockSpec((_SRC_BLK, h), lambda i: (i, 0)),
            pl.BlockSpec((h, d), lambda i: (0, 0)),
            pl.BlockSpec((_SRC_BLK, d), lambda i: (i, 0)),
            pl.BlockSpec((_SRC_BLK, d), lambda i: (i, 0)),
            pl.BlockSpec((1, 3), lambda i: (0, 0)),
        ],
        out_specs=pl.BlockSpec((_SRC_BLK, d), lambda i: (i, 0)),
        out_shape=jax.ShapeDtypeStruct((n, d), jnp.float32),
    )(hg_src, t, x1, x0, att)


def kernel(pois_embs, HG_poi_src, HG_poi_tar, layer_attention):
    att2d = layer_attention.reshape(1, -1)
    x0 = pois_embs
    t1 = _tar_mm(HG_poi_tar, x0)
    x1 = _src_mm(HG_poi_src, t1, x0)
    t2 = _tar_mm(HG_poi_tar, x1)
    return _src_final(HG_poi_src, t2, x1, x0, att2d)
